# baseline (speedup 1.0000x reference)
import jax
import jax.numpy as jnp
from jax.experimental import pallas as pl
from jax.experimental.pallas import tpu as pltpu

_OT = 256


def _matmul_block(x_ref, m_ref, o_ref):
    o_ref[...] = jax.lax.dot_general(
        x_ref[...],
        m_ref[...].astype(jnp.bfloat16),
        dimension_numbers=(((1,), (1,)), ((), ())),
        preferred_element_type=jnp.float32,
    )


def kernel(x, M):
    B, C, N = x.shape
    O = M.shape[0]
    BC = B * C
    x2 = x.reshape(BC, N).astype(jnp.bfloat16)

    y = pl.pallas_call(
        _matmul_block,
        grid=(pl.cdiv(O, _OT),),
        in_specs=[
            pl.BlockSpec((BC, N), lambda i: (0, 0)),
            pl.BlockSpec((_OT, N), lambda i: (i, 0)),
        ],
        out_specs=pl.BlockSpec((BC, _OT), lambda i: (0, i)),
        out_shape=jax.ShapeDtypeStruct((BC, O), jnp.float32),
        compiler_params=pltpu.CompilerParams(
            dimension_semantics=("parallel",)),
    )(x2, M)
    return y.reshape(B, C, O)


# speedup vs baseline: 1.0068x; 1.0068x over previous
import jax
import jax.numpy as jnp
from jax.experimental import pallas as pl
from jax.experimental.pallas import tpu as pltpu

_OT = 384


def _matmul_block(x_ref, m_ref, o_ref):
    o_ref[...] = jax.lax.dot_general(
        x_ref[...],
        m_ref[...],
        dimension_numbers=(((1,), (1,)), ((), ())),
        preferred_element_type=jnp.float32,
    )


def kernel(x, M):
    B, C, N = x.shape
    O = M.shape[0]
    BC = B * C
    x2 = x.reshape(BC, N)

    y = pl.pallas_call(
        _matmul_block,
        grid=(pl.cdiv(O, _OT),),
        in_specs=[
            pl.BlockSpec((BC, N), lambda i: (0, 0)),
            pl.BlockSpec((_OT, N), lambda i: (i, 0)),
        ],
        out_specs=pl.BlockSpec((BC, _OT), lambda i: (0, i)),
        out_shape=jax.ShapeDtypeStruct((BC, O), jnp.float32),
        compiler_params=pltpu.CompilerParams(
            dimension_semantics=("parallel",)),
    )(x2, M)
    return y.reshape(B, C, O)
